# baseline (speedup 1.0000x reference)
import jax
import jax.numpy as jnp
from jax import lax
from jax.experimental import pallas as pl
from jax.experimental.pallas import tpu as pltpu
from jax.experimental.pallas import tpu_sc as plsc

B = 1024
DIM = 128
R = 2
NBR = 16

HB = B
NW = 32
N0 = HB * R // NW
N1 = HB * R * NBR // NW
N2 = HB * R * NBR * NBR // NW
CHUNK = 128
NCH1 = N1 // CHUNK
NCH2 = N2 // CHUNK
OPC = CHUNK // NBR


def _sc_body(e0_h, e1_h, e2_h, tab_h, v0_h, v1_h, s2_h,
             idx_v, buf_a, buf_b, buf_c, buf_d, stage_a, stage_b,
             sem_a, sem_b, sem_c, sem_d, fsem_a, fsem_b):
    wid = lax.axis_index("s") * 2 + lax.axis_index("c")
    zero = jnp.zeros((16,), jnp.float32)

    bufs = [buf_a, buf_b, buf_c, buf_d]
    sems = [sem_a, sem_b, sem_c, sem_d]

    pltpu.sync_copy(e0_h.at[pl.ds(wid * N0, N0)], idx_v.at[pl.ds(0, N0)])
    pltpu.async_copy(tab_h.at[idx_v.at[pl.ds(0, N0)]],
                     buf_a.at[pl.ds(0, N0)], sem_a).wait()
    pltpu.sync_copy(buf_a.at[pl.ds(0, N0)], v0_h.at[pl.ds(wid * N0, N0)])

    pltpu.sync_copy(e1_h.at[pl.ds(wid * N1, N1)], idx_v.at[pl.ds(0, N1)])
    pend = [pltpu.async_copy(tab_h.at[idx_v.at[pl.ds(j * CHUNK, CHUNK)]],
                             bufs[j], sems[j]) for j in range(4)]
    for j in range(NCH1):
        pend[j % 4].wait()
        pltpu.sync_copy(bufs[j % 4],
                        v1_h.at[pl.ds(wid * N1 + j * CHUNK, CHUNK)])
        if j + 4 < NCH1:
            pend[j % 4] = pltpu.async_copy(
                tab_h.at[idx_v.at[pl.ds((j + 4) * CHUNK, CHUNK)]],
                bufs[j % 4], sems[j % 4])

    pltpu.sync_copy(e2_h.at[pl.ds(wid * N2, N2)], idx_v)
    for j in range(4):
        pltpu.async_copy(tab_h.at[idx_v.at[pl.ds(j * CHUNK, CHUNK)]],
                         bufs[j], sems[j])

    def do_chunk(c, buf, sem, stage, fsem):
        pltpu.make_async_copy(tab_h.at[pl.ds(0, CHUNK)], buf, sem).wait()

        @pl.when(c >= 2)
        def _():
            pltpu.make_async_copy(stage, s2_h.at[pl.ds(0, OPC)], fsem).wait()

        def obody(o, _):
            base = o * NBR
            accs = [zero] * 8
            for r in range(NBR):
                for k in range(8):
                    accs[k] = accs[k] + buf[base + r, pl.ds(k * 16, 16)]
            for k in range(8):
                stage[o, pl.ds(k * 16, 16)] = accs[k]
            return 0

        lax.fori_loop(0, OPC, obody, 0)
        pltpu.async_copy(stage, s2_h.at[pl.ds(wid * (N2 // NBR) + c * OPC,
                                              OPC)], fsem)

        @pl.when(c + 4 < NCH2)
        def _():
            pltpu.async_copy(tab_h.at[idx_v.at[pl.ds((c + 4) * CHUNK, CHUNK)]],
                             buf, sem)

    def qbody(g, _):
        c0 = 4 * g
        do_chunk(c0, buf_a, sem_a, stage_a, fsem_a)
        do_chunk(c0 + 1, buf_b, sem_b, stage_b, fsem_b)
        do_chunk(c0 + 2, buf_c, sem_c, stage_a, fsem_a)
        do_chunk(c0 + 3, buf_d, sem_d, stage_b, fsem_b)
        return 0

    lax.fori_loop(0, NCH2 // 4, qbody, 0)

    pltpu.make_async_copy(stage_a, s2_h.at[pl.ds(0, OPC)], fsem_a).wait()
    pltpu.make_async_copy(stage_b, s2_h.at[pl.ds(0, OPC)], fsem_b).wait()


def _sc_gather(e0f, e1f, e2f, table):
    mesh = plsc.VectorSubcoreMesh(core_axis_name="c", subcore_axis_name="s")
    f = pl.kernel(
        _sc_body,
        out_type=[
            jax.ShapeDtypeStruct((HB * R, DIM), jnp.float32),
            jax.ShapeDtypeStruct((HB * R * NBR, DIM), jnp.float32),
            jax.ShapeDtypeStruct((HB * R * NBR, DIM), jnp.float32),
        ],
        mesh=mesh,
        scratch_types=[
            pltpu.VMEM((N2,), jnp.int32),
            pltpu.VMEM((CHUNK, DIM), jnp.float32),
            pltpu.VMEM((CHUNK, DIM), jnp.float32),
            pltpu.VMEM((CHUNK, DIM), jnp.float32),
            pltpu.VMEM((CHUNK, DIM), jnp.float32),
            pltpu.VMEM((OPC, DIM), jnp.float32),
            pltpu.VMEM((OPC, DIM), jnp.float32),
            pltpu.SemaphoreType.DMA,
            pltpu.SemaphoreType.DMA,
            pltpu.SemaphoreType.DMA,
            pltpu.SemaphoreType.DMA,
            pltpu.SemaphoreType.DMA,
            pltpu.SemaphoreType.DMA,
        ],
    )
    return f(e0f, e1f, e2f, table)


def _matmul_t(x, w, prec):
    return lax.dot_general(x, w, (((1,), (1,)), ((), ())), precision=prec)


def _tc_body(re_ref, v0_ref, v1_ref, s2_ref, w_ref, b_ref, out_ref):
    u = v1_ref.shape[0] // (R * NBR)
    v1 = v1_ref[...]
    w = w_ref[...]
    bb = b_ref[...]
    prec = lax.Precision.DEFAULT

    s1 = v1 + s2_ref[...] * (1.0 / NBR)
    h1 = jnp.maximum(_matmul_t(s1, w, prec) + bb, 0.0)

    agg1 = jnp.sum(v1.reshape(u * R, NBR, DIM), axis=1) * (1.0 / NBR)
    h0 = jnp.maximum(_matmul_t(v0_ref[...] + agg1, w, prec) + bb, 0.0)

    aggh1 = jnp.sum(h1.reshape(u * R, NBR, DIM), axis=1) * (1.0 / NBR)
    o = jnp.tanh(_matmul_t(h0 + aggh1, w, prec) + bb)

    o3 = o.reshape(u, R, DIM)
    r0 = re_ref[0, 0]
    r1 = re_ref[0, 1]
    out_ref[...] = o3[:, 0, :] * r0 + o3[:, 1, :] * r1


def _tc_dense(v0, v1, s2, W, b2, re):
    grid = 16
    u = HB // grid
    return pl.pallas_call(
        _tc_body,
        grid=(grid,),
        in_specs=[
            pl.BlockSpec(memory_space=pltpu.SMEM),
            pl.BlockSpec((u * R, DIM), lambda i: (i, 0)),
            pl.BlockSpec((u * R * NBR, DIM), lambda i: (i, 0)),
            pl.BlockSpec((u * R * NBR, DIM), lambda i: (i, 0)),
            pl.BlockSpec((DIM, DIM), lambda i: (0, 0)),
            pl.BlockSpec((1, DIM), lambda i: (0, 0)),
        ],
        out_specs=pl.BlockSpec((u, DIM), lambda i: (i, 0)),
        out_shape=jax.ShapeDtypeStruct((HB, DIM), jnp.float32),
    )(re, v0, v1, s2, W, b2)


def kernel(e0, e1, e2, ent_embed, rule_embed, W, b):
    v0, v1, s2 = _sc_gather(e0.reshape(-1), e1.reshape(-1), e2.reshape(-1),
                            ent_embed)
    return _tc_dense(v0, v1, s2, W, b.reshape(1, DIM), rule_embed)


# speedup vs baseline: 1.0339x; 1.0339x over previous
import jax
import jax.numpy as jnp
from jax import lax
from jax.experimental import pallas as pl
from jax.experimental.pallas import tpu as pltpu
from jax.experimental.pallas import tpu_sc as plsc

B = 1024
DIM = 128
R = 2
NBR = 16

HB = B
NW = 32
N0 = HB * R // NW
N1 = HB * R * NBR // NW
N2 = HB * R * NBR * NBR // NW
CHUNK = 128
NCH1 = N1 // CHUNK
NCH2 = N2 // CHUNK
OPC = CHUNK // NBR


def _sc_body(e0_h, e1_h, e2_h, tab_h, v0_h, v1_h, s2_h,
             idx_v, buf_a, buf_b, buf_c, buf_d, stage_a, stage_b,
             sem_a, sem_b, sem_c, sem_d, fsem_a, fsem_b):
    wid = lax.axis_index("s") * 2 + lax.axis_index("c")
    zero = jnp.zeros((16,), jnp.float32)

    bufs = [buf_a, buf_b, buf_c, buf_d]
    sems = [sem_a, sem_b, sem_c, sem_d]

    pltpu.sync_copy(e0_h.at[pl.ds(wid * N0, N0)], idx_v.at[pl.ds(0, N0)])
    pltpu.async_copy(tab_h.at[idx_v.at[pl.ds(0, N0)]],
                     buf_a.at[pl.ds(0, N0)], sem_a).wait()
    pltpu.sync_copy(buf_a.at[pl.ds(0, N0)], v0_h.at[pl.ds(wid * N0, N0)])

    pltpu.sync_copy(e1_h.at[pl.ds(wid * N1, N1)], idx_v.at[pl.ds(0, N1)])
    pend = [pltpu.async_copy(tab_h.at[idx_v.at[pl.ds(j * CHUNK, CHUNK)]],
                             bufs[j], sems[j]) for j in range(4)]
    for j in range(NCH1):
        pend[j % 4].wait()
        pltpu.sync_copy(bufs[j % 4],
                        v1_h.at[pl.ds(wid * N1 + j * CHUNK, CHUNK)])
        if j + 4 < NCH1:
            pend[j % 4] = pltpu.async_copy(
                tab_h.at[idx_v.at[pl.ds((j + 4) * CHUNK, CHUNK)]],
                bufs[j % 4], sems[j % 4])

    pltpu.sync_copy(e2_h.at[pl.ds(wid * N2, N2)], idx_v)
    for j in range(4):
        pltpu.async_copy(tab_h.at[idx_v.at[pl.ds(j * CHUNK, CHUNK)]],
                         bufs[j], sems[j])

    def do_chunk(c, buf, sem, stage, fsem):
        pltpu.make_async_copy(tab_h.at[pl.ds(0, CHUNK)], buf, sem).wait()

        @pl.when(c >= 2)
        def _():
            pltpu.make_async_copy(stage, s2_h.at[pl.ds(0, OPC)], fsem).wait()

        def obody(o, _):
            base = o * NBR
            accs = [zero] * 8
            for r in range(NBR):
                for k in range(8):
                    accs[k] = accs[k] + buf[base + r, pl.ds(k * 16, 16)]
            for k in range(8):
                stage[o, pl.ds(k * 16, 16)] = accs[k]
            return 0

        lax.fori_loop(0, OPC, obody, 0)
        pltpu.async_copy(stage, s2_h.at[pl.ds(wid * (N2 // NBR) + c * OPC,
                                              OPC)], fsem)

        @pl.when(c + 4 < NCH2)
        def _():
            pltpu.async_copy(tab_h.at[idx_v.at[pl.ds((c + 4) * CHUNK, CHUNK)]],
                             buf, sem)

    def qbody(g, _):
        c0 = 4 * g
        do_chunk(c0, buf_a, sem_a, stage_a, fsem_a)
        do_chunk(c0 + 1, buf_b, sem_b, stage_b, fsem_b)
        do_chunk(c0 + 2, buf_c, sem_c, stage_a, fsem_a)
        do_chunk(c0 + 3, buf_d, sem_d, stage_b, fsem_b)
        return 0

    lax.fori_loop(0, NCH2 // 4, qbody, 0)

    pltpu.make_async_copy(stage_a, s2_h.at[pl.ds(0, OPC)], fsem_a).wait()
    pltpu.make_async_copy(stage_b, s2_h.at[pl.ds(0, OPC)], fsem_b).wait()


def _sc_gather(e0f, e1f, e2f, table):
    mesh = plsc.VectorSubcoreMesh(core_axis_name="c", subcore_axis_name="s")
    f = pl.kernel(
        _sc_body,
        out_type=[
            jax.ShapeDtypeStruct((HB * R, DIM), jnp.float32),
            jax.ShapeDtypeStruct((HB * R * NBR, DIM), jnp.float32),
            jax.ShapeDtypeStruct((HB * R * NBR, DIM), jnp.float32),
        ],
        mesh=mesh,
        scratch_types=[
            pltpu.VMEM((N2,), jnp.int32),
            pltpu.VMEM((CHUNK, DIM), jnp.float32),
            pltpu.VMEM((CHUNK, DIM), jnp.float32),
            pltpu.VMEM((CHUNK, DIM), jnp.float32),
            pltpu.VMEM((CHUNK, DIM), jnp.float32),
            pltpu.VMEM((OPC, DIM), jnp.float32),
            pltpu.VMEM((OPC, DIM), jnp.float32),
            pltpu.SemaphoreType.DMA,
            pltpu.SemaphoreType.DMA,
            pltpu.SemaphoreType.DMA,
            pltpu.SemaphoreType.DMA,
            pltpu.SemaphoreType.DMA,
            pltpu.SemaphoreType.DMA,
        ],
    )
    return f(e0f, e1f, e2f, table)


def _matmul_t(x, w, prec):
    return lax.dot_general(x, w, (((1,), (1,)), ((), ())), precision=prec)


def _tc_body(re_ref, v0_ref, v1_ref, s2_ref, w_ref, b_ref, out_ref):
    u = v1_ref.shape[0] // (R * NBR)
    v1 = v1_ref[...]
    w = w_ref[...]
    bb = b_ref[...]
    prec = lax.Precision.DEFAULT

    s1 = v1 + s2_ref[...] * (1.0 / NBR)
    h1 = jnp.maximum(_matmul_t(s1, w, prec) + bb, 0.0)

    agg1 = jnp.sum(v1.reshape(u * R, NBR, DIM), axis=1) * (1.0 / NBR)
    h0 = jnp.maximum(_matmul_t(v0_ref[...] + agg1, w, prec) + bb, 0.0)

    aggh1 = jnp.sum(h1.reshape(u * R, NBR, DIM), axis=1) * (1.0 / NBR)
    o = jnp.tanh(_matmul_t(h0 + aggh1, w, prec) + bb)

    o3 = o.reshape(u, R, DIM)
    r0 = re_ref[0, 0]
    r1 = re_ref[0, 1]
    out_ref[...] = o3[:, 0, :] * r0 + o3[:, 1, :] * r1


def _tc_dense(v0, v1, s2, W, b2, re):
    grid = 4
    u = HB // grid
    return pl.pallas_call(
        _tc_body,
        grid=(grid,),
        in_specs=[
            pl.BlockSpec(memory_space=pltpu.SMEM),
            pl.BlockSpec((u * R, DIM), lambda i: (i, 0)),
            pl.BlockSpec((u * R * NBR, DIM), lambda i: (i, 0)),
            pl.BlockSpec((u * R * NBR, DIM), lambda i: (i, 0)),
            pl.BlockSpec((DIM, DIM), lambda i: (0, 0)),
            pl.BlockSpec((1, DIM), lambda i: (0, 0)),
        ],
        out_specs=pl.BlockSpec((u, DIM), lambda i: (i, 0)),
        out_shape=jax.ShapeDtypeStruct((HB, DIM), jnp.float32),
    )(re, v0, v1, s2, W, b2)


def kernel(e0, e1, e2, ent_embed, rule_embed, W, b):
    v0, v1, s2 = _sc_gather(e0.reshape(-1), e1.reshape(-1), e2.reshape(-1),
                            ent_embed)
    return _tc_dense(v0, v1, s2, W, b.reshape(1, DIM), rule_embed)


# speedup vs baseline: 1.0348x; 1.0009x over previous
import jax
import jax.numpy as jnp
from jax import lax
from jax.experimental import pallas as pl
from jax.experimental.pallas import tpu as pltpu
from jax.experimental.pallas import tpu_sc as plsc

B = 1024
DIM = 128
R = 2
NBR = 16

HB = B
NW = 32
N0 = HB * R // NW
N1 = HB * R * NBR // NW
N2 = HB * R * NBR * NBR // NW
CHUNK = 128
NCH1 = N1 // CHUNK
NCH2 = N2 // CHUNK
OPC = CHUNK // NBR


def _sc_body(e0_h, e1_h, e2_h, tab_h, v0_h, v1_h, s2_h,
             idx_v, buf_a, buf_b, buf_c, buf_d, stage_a, stage_b,
             sem_a, sem_b, sem_c, sem_d, fsem_a, fsem_b):
    wid = lax.axis_index("s") * 2 + lax.axis_index("c")
    zero = jnp.zeros((16,), jnp.float32)

    bufs = [buf_a, buf_b, buf_c, buf_d]
    sems = [sem_a, sem_b, sem_c, sem_d]

    pltpu.sync_copy(e0_h.at[pl.ds(wid * N0, N0)], idx_v.at[pl.ds(0, N0)])
    pltpu.async_copy(tab_h.at[idx_v.at[pl.ds(0, N0)]],
                     buf_a.at[pl.ds(0, N0)], sem_a).wait()
    pltpu.sync_copy(buf_a.at[pl.ds(0, N0)], v0_h.at[pl.ds(wid * N0, N0)])

    pltpu.sync_copy(e1_h.at[pl.ds(wid * N1, N1)], idx_v.at[pl.ds(0, N1)])
    osems = [fsem_a, fsem_b, fsem_a, fsem_b]
    pend = [pltpu.async_copy(tab_h.at[idx_v.at[pl.ds(j * CHUNK, CHUNK)]],
                             bufs[j], sems[j]) for j in range(4)]
    outp = [None] * 4
    for j in range(NCH1):
        if 0 < j <= NCH1 - 4:
            k = (j - 1) % 4
            outp[k].wait()
            pend[k] = pltpu.async_copy(
                tab_h.at[idx_v.at[pl.ds((j + 3) * CHUNK, CHUNK)]],
                bufs[k], sems[k])
        pend[j % 4].wait()
        outp[j % 4] = pltpu.async_copy(
            bufs[j % 4], v1_h.at[pl.ds(wid * N1 + j * CHUNK, CHUNK)],
            osems[j % 4])
    for j in range(NCH1 - 4, NCH1):
        outp[j % 4].wait()

    pltpu.sync_copy(e2_h.at[pl.ds(wid * N2, N2)], idx_v)
    for j in range(4):
        pltpu.async_copy(tab_h.at[idx_v.at[pl.ds(j * CHUNK, CHUNK)]],
                         bufs[j], sems[j])

    def do_chunk(c, buf, sem, stage, fsem):
        pltpu.make_async_copy(tab_h.at[pl.ds(0, CHUNK)], buf, sem).wait()

        @pl.when(c >= 2)
        def _():
            pltpu.make_async_copy(stage, s2_h.at[pl.ds(0, OPC)], fsem).wait()

        def obody(o, _):
            base = o * NBR
            accs = [zero] * 8
            for r in range(NBR):
                for k in range(8):
                    accs[k] = accs[k] + buf[base + r, pl.ds(k * 16, 16)]
            for k in range(8):
                stage[o, pl.ds(k * 16, 16)] = accs[k]
            return 0

        lax.fori_loop(0, OPC, obody, 0)
        pltpu.async_copy(stage, s2_h.at[pl.ds(wid * (N2 // NBR) + c * OPC,
                                              OPC)], fsem)

        @pl.when(c + 4 < NCH2)
        def _():
            pltpu.async_copy(tab_h.at[idx_v.at[pl.ds((c + 4) * CHUNK, CHUNK)]],
                             buf, sem)

    def qbody(g, _):
        c0 = 4 * g
        do_chunk(c0, buf_a, sem_a, stage_a, fsem_a)
        do_chunk(c0 + 1, buf_b, sem_b, stage_b, fsem_b)
        do_chunk(c0 + 2, buf_c, sem_c, stage_a, fsem_a)
        do_chunk(c0 + 3, buf_d, sem_d, stage_b, fsem_b)
        return 0

    lax.fori_loop(0, NCH2 // 4, qbody, 0)

    pltpu.make_async_copy(stage_a, s2_h.at[pl.ds(0, OPC)], fsem_a).wait()
    pltpu.make_async_copy(stage_b, s2_h.at[pl.ds(0, OPC)], fsem_b).wait()


def _sc_gather(e0f, e1f, e2f, table):
    mesh = plsc.VectorSubcoreMesh(core_axis_name="c", subcore_axis_name="s")
    f = pl.kernel(
        _sc_body,
        out_type=[
            jax.ShapeDtypeStruct((HB * R, DIM), jnp.float32),
            jax.ShapeDtypeStruct((HB * R * NBR, DIM), jnp.float32),
            jax.ShapeDtypeStruct((HB * R * NBR, DIM), jnp.float32),
        ],
        mesh=mesh,
        scratch_types=[
            pltpu.VMEM((N2,), jnp.int32),
            pltpu.VMEM((CHUNK, DIM), jnp.float32),
            pltpu.VMEM((CHUNK, DIM), jnp.float32),
            pltpu.VMEM((CHUNK, DIM), jnp.float32),
            pltpu.VMEM((CHUNK, DIM), jnp.float32),
            pltpu.VMEM((OPC, DIM), jnp.float32),
            pltpu.VMEM((OPC, DIM), jnp.float32),
            pltpu.SemaphoreType.DMA,
            pltpu.SemaphoreType.DMA,
            pltpu.SemaphoreType.DMA,
            pltpu.SemaphoreType.DMA,
            pltpu.SemaphoreType.DMA,
            pltpu.SemaphoreType.DMA,
        ],
    )
    return f(e0f, e1f, e2f, table)


def _matmul_t(x, w, prec):
    return lax.dot_general(x, w, (((1,), (1,)), ((), ())), precision=prec)


def _tc_body(re_ref, v0_ref, v1_ref, s2_ref, w_ref, b_ref, out_ref):
    u = v1_ref.shape[0] // (R * NBR)
    v1 = v1_ref[...]
    w = w_ref[...]
    bb = b_ref[...]
    prec = lax.Precision.DEFAULT

    s1 = v1 + s2_ref[...] * (1.0 / NBR)
    h1 = jnp.maximum(_matmul_t(s1, w, prec) + bb, 0.0)

    agg1 = jnp.sum(v1.reshape(u * R, NBR, DIM), axis=1) * (1.0 / NBR)
    h0 = jnp.maximum(_matmul_t(v0_ref[...] + agg1, w, prec) + bb, 0.0)

    aggh1 = jnp.sum(h1.reshape(u * R, NBR, DIM), axis=1) * (1.0 / NBR)
    o = jnp.tanh(_matmul_t(h0 + aggh1, w, prec) + bb)

    o3 = o.reshape(u, R, DIM)
    r0 = re_ref[0, 0]
    r1 = re_ref[0, 1]
    out_ref[...] = o3[:, 0, :] * r0 + o3[:, 1, :] * r1


def _tc_dense(v0, v1, s2, W, b2, re):
    grid = 4
    u = HB // grid
    return pl.pallas_call(
        _tc_body,
        grid=(grid,),
        in_specs=[
            pl.BlockSpec(memory_space=pltpu.SMEM),
            pl.BlockSpec((u * R, DIM), lambda i: (i, 0)),
            pl.BlockSpec((u * R * NBR, DIM), lambda i: (i, 0)),
            pl.BlockSpec((u * R * NBR, DIM), lambda i: (i, 0)),
            pl.BlockSpec((DIM, DIM), lambda i: (0, 0)),
            pl.BlockSpec((1, DIM), lambda i: (0, 0)),
        ],
        out_specs=pl.BlockSpec((u, DIM), lambda i: (i, 0)),
        out_shape=jax.ShapeDtypeStruct((HB, DIM), jnp.float32),
    )(re, v0, v1, s2, W, b2)


def kernel(e0, e1, e2, ent_embed, rule_embed, W, b):
    v0, v1, s2 = _sc_gather(e0.reshape(-1), e1.reshape(-1), e2.reshape(-1),
                            ent_embed)
    return _tc_dense(v0, v1, s2, W, b.reshape(1, DIM), rule_embed)


# speedup vs baseline: 1.0448x; 1.0096x over previous
import jax
import jax.numpy as jnp
from jax import lax
from jax.experimental import pallas as pl
from jax.experimental.pallas import tpu as pltpu
from jax.experimental.pallas import tpu_sc as plsc

B = 1024
DIM = 128
R = 2
NBR = 16

HB = B
NW = 32
N0 = HB * R // NW
N1 = HB * R * NBR // NW
N2 = HB * R * NBR * NBR // NW
CHUNK = 128
NCH1 = N1 // CHUNK
NCH2 = N2 // CHUNK
OPC = CHUNK // NBR


def _sc_body(e0_h, e1_h, e2_h, tab_h, v0_h, v1_h, s2_h,
             idx_v, idx01_v, buf_a, buf_b, buf_c, buf_d, stage_a, stage_b,
             sem_a, sem_b, sem_c, sem_d, fsem_a, fsem_b):
    wid = lax.axis_index("s") * 2 + lax.axis_index("c")
    zero = jnp.zeros((16,), jnp.float32)

    bufs = [buf_a, buf_b, buf_c, buf_d]
    sems = [sem_a, sem_b, sem_c, sem_d]

    ld0 = pltpu.async_copy(e0_h.at[pl.ds(wid * N0, N0)],
                           idx01_v.at[pl.ds(0, N0)], fsem_a)
    ld1 = pltpu.async_copy(e1_h.at[pl.ds(wid * N1, N1)],
                           idx01_v.at[pl.ds(N0, N1)], fsem_b)
    ld2 = pltpu.async_copy(e2_h.at[pl.ds(wid * N2, N2)], idx_v, sem_b)

    ld0.wait()
    g0 = pltpu.async_copy(tab_h.at[idx01_v.at[pl.ds(0, N0)]],
                          buf_d.at[pl.ds(0, N0)], sem_a)
    ld1.wait()
    ld2.wait()
    g0.wait()
    v0out = pltpu.async_copy(buf_d.at[pl.ds(0, N0)],
                             v0_h.at[pl.ds(wid * N0, N0)], fsem_a)

    osems = [fsem_b, fsem_a, fsem_b, fsem_a]
    pend = [pltpu.async_copy(
        tab_h.at[idx01_v.at[pl.ds(N0 + j * CHUNK, CHUNK)]],
        bufs[j], sems[j]) for j in range(3)]
    v0out.wait()
    pend.append(pltpu.async_copy(
        tab_h.at[idx01_v.at[pl.ds(N0 + 3 * CHUNK, CHUNK)]], buf_d, sem_d))
    outp = [None] * 4
    for j in range(NCH1):
        if 0 < j <= NCH1 - 4:
            k = (j - 1) % 4
            outp[k].wait()
            pend[k] = pltpu.async_copy(
                tab_h.at[idx01_v.at[pl.ds(N0 + (j + 3) * CHUNK, CHUNK)]],
                bufs[k], sems[k])
        pend[j % 4].wait()
        outp[j % 4] = pltpu.async_copy(
            bufs[j % 4], v1_h.at[pl.ds(wid * N1 + j * CHUNK, CHUNK)],
            osems[j % 4])
    for j in range(NCH1 - 4, NCH1):
        outp[j % 4].wait()

    for j in range(4):
        pltpu.async_copy(tab_h.at[idx_v.at[pl.ds(j * CHUNK, CHUNK)]],
                         bufs[j], sems[j])

    def do_chunk(c, buf, sem, stage, fsem):
        pltpu.make_async_copy(tab_h.at[pl.ds(0, CHUNK)], buf, sem).wait()

        @pl.when(c >= 2)
        def _():
            pltpu.make_async_copy(stage, s2_h.at[pl.ds(0, OPC)], fsem).wait()

        def obody(o, _):
            base = o * NBR
            accs = [zero] * 8
            for r in range(NBR):
                for k in range(8):
                    accs[k] = accs[k] + buf[base + r, pl.ds(k * 16, 16)]
            for k in range(8):
                stage[o, pl.ds(k * 16, 16)] = accs[k]
            return 0

        lax.fori_loop(0, OPC, obody, 0)
        pltpu.async_copy(stage, s2_h.at[pl.ds(wid * (N2 // NBR) + c * OPC,
                                              OPC)], fsem)

        @pl.when(c + 4 < NCH2)
        def _():
            pltpu.async_copy(tab_h.at[idx_v.at[pl.ds((c + 4) * CHUNK, CHUNK)]],
                             buf, sem)

    def qbody(g, _):
        c0 = 4 * g
        do_chunk(c0, buf_a, sem_a, stage_a, fsem_a)
        do_chunk(c0 + 1, buf_b, sem_b, stage_b, fsem_b)
        do_chunk(c0 + 2, buf_c, sem_c, stage_a, fsem_a)
        do_chunk(c0 + 3, buf_d, sem_d, stage_b, fsem_b)
        return 0

    lax.fori_loop(0, NCH2 // 4, qbody, 0)

    pltpu.make_async_copy(stage_a, s2_h.at[pl.ds(0, OPC)], fsem_a).wait()
    pltpu.make_async_copy(stage_b, s2_h.at[pl.ds(0, OPC)], fsem_b).wait()


def _sc_gather(e0f, e1f, e2f, table):
    mesh = plsc.VectorSubcoreMesh(core_axis_name="c", subcore_axis_name="s")
    f = pl.kernel(
        _sc_body,
        out_type=[
            jax.ShapeDtypeStruct((HB * R, DIM), jnp.float32),
            jax.ShapeDtypeStruct((HB * R * NBR, DIM), jnp.float32),
            jax.ShapeDtypeStruct((HB * R * NBR, DIM), jnp.float32),
        ],
        mesh=mesh,
        scratch_types=[
            pltpu.VMEM((N2,), jnp.int32),
            pltpu.VMEM((N0 + N1,), jnp.int32),
            pltpu.VMEM((CHUNK, DIM), jnp.float32),
            pltpu.VMEM((CHUNK, DIM), jnp.float32),
            pltpu.VMEM((CHUNK, DIM), jnp.float32),
            pltpu.VMEM((CHUNK, DIM), jnp.float32),
            pltpu.VMEM((OPC, DIM), jnp.float32),
            pltpu.VMEM((OPC, DIM), jnp.float32),
            pltpu.SemaphoreType.DMA,
            pltpu.SemaphoreType.DMA,
            pltpu.SemaphoreType.DMA,
            pltpu.SemaphoreType.DMA,
            pltpu.SemaphoreType.DMA,
            pltpu.SemaphoreType.DMA,
        ],
    )
    return f(e0f, e1f, e2f, table)


def _matmul_t(x, w, prec):
    return lax.dot_general(x, w, (((1,), (1,)), ((), ())), precision=prec)


def _tc_body(re_ref, v0_ref, v1_ref, s2_ref, w_ref, b_ref, out_ref):
    u = v1_ref.shape[0] // (R * NBR)
    v1 = v1_ref[...]
    w = w_ref[...]
    bb = b_ref[...]
    prec = lax.Precision.DEFAULT

    s1 = v1 + s2_ref[...] * (1.0 / NBR)
    h1 = jnp.maximum(_matmul_t(s1, w, prec) + bb, 0.0)

    agg1 = jnp.sum(v1.reshape(u * R, NBR, DIM), axis=1) * (1.0 / NBR)
    h0 = jnp.maximum(_matmul_t(v0_ref[...] + agg1, w, prec) + bb, 0.0)

    aggh1 = jnp.sum(h1.reshape(u * R, NBR, DIM), axis=1) * (1.0 / NBR)
    o = jnp.tanh(_matmul_t(h0 + aggh1, w, prec) + bb)

    o3 = o.reshape(u, R, DIM)
    r0 = re_ref[0, 0]
    r1 = re_ref[0, 1]
    out_ref[...] = o3[:, 0, :] * r0 + o3[:, 1, :] * r1


def _tc_dense(v0, v1, s2, W, b2, re):
    grid = 4
    u = HB // grid
    return pl.pallas_call(
        _tc_body,
        grid=(grid,),
        in_specs=[
            pl.BlockSpec(memory_space=pltpu.SMEM),
            pl.BlockSpec((u * R, DIM), lambda i: (i, 0)),
            pl.BlockSpec((u * R * NBR, DIM), lambda i: (i, 0)),
            pl.BlockSpec((u * R * NBR, DIM), lambda i: (i, 0)),
            pl.BlockSpec((DIM, DIM), lambda i: (0, 0)),
            pl.BlockSpec((1, DIM), lambda i: (0, 0)),
        ],
        out_specs=pl.BlockSpec((u, DIM), lambda i: (i, 0)),
        out_shape=jax.ShapeDtypeStruct((HB, DIM), jnp.float32),
    )(re, v0, v1, s2, W, b2)


def kernel(e0, e1, e2, ent_embed, rule_embed, W, b):
    v0, v1, s2 = _sc_gather(e0.reshape(-1), e1.reshape(-1), e2.reshape(-1),
                            ent_embed)
    return _tc_dense(v0, v1, s2, W, b.reshape(1, DIM), rule_embed)
